# Initial kernel scaffold; baseline (speedup 1.0000x reference)
import jax
import jax.numpy as jnp
from jax.experimental import pallas as pl


def kernel(x, edge_index, W1, b1, W2, b2):
    raise NotImplementedError("write your pallas kernel here")


# speedup vs baseline: 22.6269x; 22.6269x over previous
import functools

import jax
import jax.numpy as jnp
from jax import lax
from jax.experimental import pallas as pl
from jax.experimental.pallas import tpu as pltpu
from jax.experimental.pallas import tpu_sc as plsc

N = 10000
HID = 32
DEGW = 16

NC = 2
NS = 16
NW = NC * NS
K = 128

N_PAD = 10240
RPT = N_PAD // NS

_mesh = plsc.VectorSubcoreMesh(core_axis_name="c", subcore_axis_name="s")


def _zero_rows(ref, nrows, ncols):
    zero = jnp.zeros((16,), jnp.float32)

    def body(i, _):
        for c in range(ncols // 16):
            ref[i, pl.ds(c * 16, 16)] = zero
        return 0

    lax.fori_loop(0, nrows, body, 0)


def _make_deg_kernel(nchunk):
    @functools.partial(
        pl.kernel,
        out_type=jax.ShapeDtypeStruct((NC, N_PAD, DEGW), jnp.float32),
        mesh=_mesh,
        scratch_types=[
            pltpu.VMEM((nchunk, K), jnp.int32),
            pltpu.VMEM((K, DEGW), jnp.float32),
            pltpu.VMEM((RPT, DEGW), jnp.float32),
            pltpu.VMEM_SHARED((N_PAD, DEGW), jnp.float32),
        ],
        compiler_params=pltpu.CompilerParams(use_tc_tiling_on_sc=False),
    )
    def deg_kernel(col_hbm, degp_hbm, col_v, ones_v, zrow_v, deg_sh):
        cid = lax.axis_index("c")
        sid = lax.axis_index("s")
        wid = sid * NC + cid

        pltpu.sync_copy(col_hbm.at[wid], col_v)

        one = jnp.full((16,), 1.0, jnp.float32)

        def fill_ones(i, _):
            ones_v[i, :] = one
            return 0

        lax.fori_loop(0, K, fill_ones, 0)
        _zero_rows(zrow_v, RPT, DEGW)

        pltpu.sync_copy(zrow_v, deg_sh.at[pl.ds(sid * RPT, RPT)])
        plsc.subcore_barrier()

        def chunk(j, _):
            pltpu.sync_copy(ones_v, deg_sh.at[col_v.at[j]], add=True)
            return 0

        lax.fori_loop(0, nchunk, chunk, 0)
        plsc.subcore_barrier()

        pltpu.sync_copy(
            deg_sh.at[pl.ds(sid * RPT, RPT)],
            degp_hbm.at[cid, pl.ds(sid * RPT, RPT)],
        )

    return deg_kernel


def _make_agg_kernel(nchunk):
    @functools.partial(
        pl.kernel,
        out_type=jax.ShapeDtypeStruct((NC, N_PAD, HID), jnp.float32),
        mesh=_mesh,
        scratch_types=[
            pltpu.VMEM((nchunk, K), jnp.int32),
            pltpu.VMEM((nchunk, K), jnp.int32),
            pltpu.VMEM((K, HID), jnp.float32),
            pltpu.VMEM((K, HID), jnp.float32),
            pltpu.VMEM((RPT, HID), jnp.float32),
            pltpu.VMEM_SHARED((N_PAD, HID), jnp.float32),
            pltpu.SemaphoreType.DMA,
            pltpu.SemaphoreType.DMA,
        ],
        compiler_params=pltpu.CompilerParams(use_tc_tiling_on_sc=False),
    )
    def agg_kernel(feat_hbm, row_hbm, col_hbm, aggp_hbm,
                   row_v, col_v, buf0, buf1, zrow_v, agg_sh, sem0, sem1):
        cid = lax.axis_index("c")
        sid = lax.axis_index("s")
        wid = sid * NC + cid

        pltpu.sync_copy(row_hbm.at[wid], row_v)
        pltpu.sync_copy(col_hbm.at[wid], col_v)

        _zero_rows(zrow_v, RPT, HID)
        pltpu.sync_copy(zrow_v, agg_sh.at[pl.ds(sid * RPT, RPT)])
        plsc.subcore_barrier()

        pltpu.async_copy(feat_hbm.at[row_v.at[0]], buf0, sem0)
        pltpu.async_copy(feat_hbm.at[row_v.at[1]], buf1, sem1)

        def step(h, _):
            g = 2 * h
            pltpu.make_async_copy(feat_hbm.at[row_v.at[g]], buf0, sem0).wait()
            pltpu.sync_copy(buf0, agg_sh.at[col_v.at[g]], add=True)
            pltpu.async_copy(
                feat_hbm.at[row_v.at[(g + 2) % nchunk]], buf0, sem0)

            pltpu.make_async_copy(
                feat_hbm.at[row_v.at[g + 1]], buf1, sem1).wait()
            pltpu.sync_copy(buf1, agg_sh.at[col_v.at[g + 1]], add=True)
            pltpu.async_copy(
                feat_hbm.at[row_v.at[(g + 3) % nchunk]], buf1, sem1)
            return 0

        lax.fori_loop(0, nchunk // 2, step, 0)

        pltpu.make_async_copy(feat_hbm.at[row_v.at[0]], buf0, sem0).wait()
        pltpu.make_async_copy(feat_hbm.at[row_v.at[1]], buf1, sem1).wait()
        plsc.subcore_barrier()

        pltpu.sync_copy(
            agg_sh.at[pl.ds(sid * RPT, RPT)],
            aggp_hbm.at[cid, pl.ds(sid * RPT, RPT)],
        )

    return agg_kernel


def _dinv_from_partials(degp):
    deg = degp[0, :, 0:1] + degp[1, :, 0:1]
    return jnp.where(deg > 0, lax.rsqrt(jnp.maximum(deg, 1e-12)), 0.0)


def _tc_xw_scale(x_pad, w1, degp):
    def body(x_ref, w_ref, degp_ref, out_ref):
        dinv = _dinv_from_partials(degp_ref[...])
        xw = jnp.dot(x_ref[...], w_ref[...],
                     preferred_element_type=jnp.float32)
        out_ref[...] = xw * dinv

    return pl.pallas_call(
        body,
        out_shape=jax.ShapeDtypeStruct((N_PAD, HID), jnp.float32),
    )(x_pad, w1, degp)


def _tc_mid(aggp, degp, b1):
    def body(aggp_ref, degp_ref, b1_ref, out_ref):
        dinv = _dinv_from_partials(degp_ref[...])
        agg = aggp_ref[0] + aggp_ref[1]
        h = jnp.maximum(agg * dinv + b1_ref[...], 0.0)
        rows = lax.broadcasted_iota(jnp.int32, (N_PAD, 1), 0)
        out_ref[...] = jnp.where(rows < N, h * dinv, 0.0)

    return pl.pallas_call(
        body,
        out_shape=jax.ShapeDtypeStruct((N_PAD, HID), jnp.float32),
    )(aggp, degp, b1)


def _tc_final(aggp, degp, w2, b2):
    def body(aggp_ref, degp_ref, w2_ref, b2_ref, out_ref):
        dinv = _dinv_from_partials(degp_ref[...])
        agg = (aggp_ref[0] + aggp_ref[1]) * dinv
        y = jnp.dot(agg[:N, :], w2_ref[...],
                    preferred_element_type=jnp.float32) + b2_ref[...]
        m = jnp.max(y, axis=1, keepdims=True)
        s = y - m
        lse = jnp.log(jnp.sum(jnp.exp(s), axis=1, keepdims=True))
        out_ref[...] = s - lse

    return pl.pallas_call(
        body,
        out_shape=jax.ShapeDtypeStruct((N, 128), jnp.float32),
    )(aggp, degp, w2, b2)


def kernel(x, edge_index, W1, b1, W2, b2):
    e = edge_index.shape[1]
    nchunk = -(-e // (NW * K))
    if nchunk % 2:
        nchunk += 1
    e_pad = NW * nchunk * K

    row = edge_index[0].astype(jnp.int32)
    col = edge_index[1].astype(jnp.int32)
    pad = jnp.full((e_pad - e,), N, jnp.int32)
    row_b = jnp.concatenate([row, pad]).reshape(NW, nchunk, K)
    col_b = jnp.concatenate([col, pad]).reshape(NW, nchunk, K)

    x_pad = jnp.pad(x, ((0, N_PAD - x.shape[0]), (0, 0)))

    degp = _make_deg_kernel(nchunk)(col_b)
    t1s = _tc_xw_scale(x_pad, W1, degp)
    agg1p = _make_agg_kernel(nchunk)(t1s, row_b, col_b)
    h_s = _tc_mid(agg1p, degp, b1)
    agg2p = _make_agg_kernel(nchunk)(h_s, row_b, col_b)
    return _tc_final(agg2p, degp, W2, b2)


# speedup vs baseline: 22.9077x; 1.0124x over previous
import functools

import jax
import jax.numpy as jnp
from jax import lax
from jax.experimental import pallas as pl
from jax.experimental.pallas import tpu as pltpu
from jax.experimental.pallas import tpu_sc as plsc

N = 10000
HID = 32
DEGW = 16

NC = 2
NS = 16
NW = NC * NS
K = 128
NBUF = 8
HALF = NBUF // 2

N_PAD = 10240
RPT = N_PAD // NS

_mesh = plsc.VectorSubcoreMesh(core_axis_name="c", subcore_axis_name="s")


def _zero_rows(ref, nrows, ncols):
    zero = jnp.zeros((16,), jnp.float32)

    def body(i, _):
        for c in range(ncols // 16):
            ref[i, pl.ds(c * 16, 16)] = zero
        return 0

    lax.fori_loop(0, nrows, body, 0)


def _make_deg_kernel(nchunk):
    @functools.partial(
        pl.kernel,
        out_type=jax.ShapeDtypeStruct((NC, N_PAD, DEGW), jnp.float32),
        mesh=_mesh,
        scratch_types=[
            pltpu.VMEM((nchunk, K), jnp.int32),
            pltpu.VMEM((K, DEGW), jnp.float32),
            pltpu.VMEM((RPT, DEGW), jnp.float32),
            pltpu.VMEM_SHARED((N_PAD, DEGW), jnp.float32),
            pltpu.SemaphoreType.DMA,
        ],
        compiler_params=pltpu.CompilerParams(use_tc_tiling_on_sc=False),
    )
    def deg_kernel(col_hbm, degp_hbm, col_v, ones_v, zrow_v, deg_sh, dsem):
        cid = lax.axis_index("c")
        sid = lax.axis_index("s")
        wid = sid * NC + cid

        pltpu.sync_copy(col_hbm.at[wid], col_v)

        one = jnp.full((16,), 1.0, jnp.float32)

        def fill_ones(i, _):
            ones_v[i, :] = one
            return 0

        lax.fori_loop(0, K, fill_ones, 0)
        _zero_rows(zrow_v, RPT, DEGW)

        pltpu.sync_copy(zrow_v, deg_sh.at[pl.ds(sid * RPT, RPT)])
        plsc.subcore_barrier()

        for j in range(8):
            pltpu.async_copy(ones_v, deg_sh.at[col_v.at[j]], dsem, add=True)

        def chunk(h, _):
            for j in range(8):
                s = 8 * h + j
                pltpu.make_async_copy(
                    ones_v, deg_sh.at[col_v.at[s]], dsem).wait()
                pltpu.async_copy(
                    ones_v, deg_sh.at[col_v.at[s + 8]], dsem, add=True)
            return 0

        lax.fori_loop(0, nchunk // 8 - 1, chunk, 0)
        for j in range(8):
            pltpu.make_async_copy(
                ones_v, deg_sh.at[col_v.at[j]], dsem).wait()
        plsc.subcore_barrier()

        pltpu.sync_copy(
            deg_sh.at[pl.ds(sid * RPT, RPT)],
            degp_hbm.at[cid, pl.ds(sid * RPT, RPT)],
        )

    return deg_kernel


def _make_agg_kernel(nchunk):
    @functools.partial(
        pl.kernel,
        out_type=jax.ShapeDtypeStruct((NC, N_PAD, HID), jnp.float32),
        mesh=_mesh,
        scratch_types=[
            pltpu.VMEM((nchunk, K), jnp.int32),
            pltpu.VMEM((nchunk, K), jnp.int32),
            [pltpu.VMEM((K, HID), jnp.float32) for _ in range(NBUF)],
            pltpu.VMEM((RPT, HID), jnp.float32),
            pltpu.VMEM_SHARED((N_PAD, HID), jnp.float32),
            [pltpu.SemaphoreType.DMA for _ in range(NBUF)],
            [pltpu.SemaphoreType.DMA for _ in range(NBUF)],
        ],
        compiler_params=pltpu.CompilerParams(use_tc_tiling_on_sc=False),
    )
    def agg_kernel(feat_hbm, row_hbm, col_hbm, aggp_hbm,
                   row_v, col_v, bufs, zrow_v, agg_sh, gsem, ssem):
        cid = lax.axis_index("c")
        sid = lax.axis_index("s")
        wid = sid * NC + cid

        pltpu.sync_copy(row_hbm.at[wid], row_v)
        pltpu.sync_copy(col_hbm.at[wid], col_v)

        _zero_rows(zrow_v, RPT, HID)
        pltpu.sync_copy(zrow_v, agg_sh.at[pl.ds(sid * RPT, RPT)])
        plsc.subcore_barrier()

        zsrc = zrow_v.at[pl.ds(0, K)]
        for j in range(HALF):
            pltpu.async_copy(
                zsrc, agg_sh.at[col_v.at[0]], ssem[HALF + j], add=True)
        for j in range(HALF):
            pltpu.async_copy(feat_hbm.at[row_v.at[j]], bufs[j], gsem[j])

        def step(h, _):
            for j in range(NBUF):
                s = NBUF * h + j
                pltpu.make_async_copy(
                    feat_hbm.at[row_v.at[s]], bufs[j], gsem[j]).wait()
                pltpu.async_copy(
                    bufs[j], agg_sh.at[col_v.at[s]], ssem[j], add=True)
                bb = (j + HALF) % NBUF
                pltpu.make_async_copy(
                    bufs[bb], agg_sh.at[col_v.at[s]], ssem[bb]).wait()
                s2 = (s + HALF) % nchunk
                pltpu.async_copy(feat_hbm.at[row_v.at[s2]], bufs[bb], gsem[bb])
            return 0

        lax.fori_loop(0, nchunk // NBUF, step, 0)

        for j in range(HALF):
            pltpu.make_async_copy(
                bufs[HALF + j], agg_sh.at[col_v.at[0]], ssem[HALF + j]).wait()
            pltpu.make_async_copy(
                feat_hbm.at[row_v.at[j]], bufs[j], gsem[j]).wait()
        plsc.subcore_barrier()

        pltpu.sync_copy(
            agg_sh.at[pl.ds(sid * RPT, RPT)],
            aggp_hbm.at[cid, pl.ds(sid * RPT, RPT)],
        )

    return agg_kernel


def _dinv_from_partials(degp):
    deg = degp[0, :, 0:1] + degp[1, :, 0:1]
    return jnp.where(deg > 0, lax.rsqrt(jnp.maximum(deg, 1e-12)), 0.0)


def _tc_xw_scale(x_pad, w1, degp):
    def body(x_ref, w_ref, degp_ref, out_ref):
        dinv = _dinv_from_partials(degp_ref[...])
        xw = jnp.dot(x_ref[...], w_ref[...],
                     preferred_element_type=jnp.float32)
        out_ref[...] = xw * dinv

    return pl.pallas_call(
        body,
        out_shape=jax.ShapeDtypeStruct((N_PAD, HID), jnp.float32),
    )(x_pad, w1, degp)


def _tc_mid(aggp, degp, b1):
    def body(aggp_ref, degp_ref, b1_ref, out_ref):
        dinv = _dinv_from_partials(degp_ref[...])
        agg = aggp_ref[0] + aggp_ref[1]
        h = jnp.maximum(agg * dinv + b1_ref[...], 0.0)
        rows = lax.broadcasted_iota(jnp.int32, (N_PAD, 1), 0)
        out_ref[...] = jnp.where(rows < N, h * dinv, 0.0)

    return pl.pallas_call(
        body,
        out_shape=jax.ShapeDtypeStruct((N_PAD, HID), jnp.float32),
    )(aggp, degp, b1)


def _tc_final(aggp, degp, w2, b2):
    def body(aggp_ref, degp_ref, w2_ref, b2_ref, out_ref):
        dinv = _dinv_from_partials(degp_ref[...])
        agg = (aggp_ref[0] + aggp_ref[1]) * dinv
        y = jnp.dot(agg[:N, :], w2_ref[...],
                    preferred_element_type=jnp.float32) + b2_ref[...]
        m = jnp.max(y, axis=1, keepdims=True)
        s = y - m
        lse = jnp.log(jnp.sum(jnp.exp(s), axis=1, keepdims=True))
        out_ref[...] = s - lse

    return pl.pallas_call(
        body,
        out_shape=jax.ShapeDtypeStruct((N, 128), jnp.float32),
    )(aggp, degp, w2, b2)


def kernel(x, edge_index, W1, b1, W2, b2):
    e = edge_index.shape[1]
    nchunk = -(-e // (NW * K))
    nchunk = -(-nchunk // NBUF) * NBUF
    e_pad = NW * nchunk * K

    row = edge_index[0].astype(jnp.int32)
    col = edge_index[1].astype(jnp.int32)
    pad = jnp.full((e_pad - e,), N, jnp.int32)
    row_b = jnp.concatenate([row, pad]).reshape(NW, nchunk, K)
    col_b = jnp.concatenate([col, pad]).reshape(NW, nchunk, K)

    x_pad = jnp.pad(x, ((0, N_PAD - x.shape[0]), (0, 0)))

    degp = _make_deg_kernel(nchunk)(col_b)
    t1s = _tc_xw_scale(x_pad, W1, degp)
    agg1p = _make_agg_kernel(nchunk)(t1s, row_b, col_b)
    h_s = _tc_mid(agg1p, degp, b1)
    agg2p = _make_agg_kernel(nchunk)(h_s, row_b, col_b)
    return _tc_final(agg2p, degp, W2, b2)


# speedup vs baseline: 41.4584x; 1.8098x over previous
import functools

import jax
import jax.numpy as jnp
from jax import lax
from jax.experimental import pallas as pl
from jax.experimental.pallas import tpu as pltpu
from jax.experimental.pallas import tpu_sc as plsc

N = 10000
HID = 32
DEGW = 16

NC = 2
NS = 16
NW = NC * NS
K = 128
NBUF = 8
HALF = NBUF // 2

N_PAD = 10240
RPT = N_PAD // NS

_mesh = plsc.VectorSubcoreMesh(core_axis_name="c", subcore_axis_name="s")


def _zero_rows(ref, nrows, ncols):
    zero = jnp.zeros((16,), jnp.float32)

    def body(i, _):
        for c in range(ncols // 16):
            ref[i, pl.ds(c * 16, 16)] = zero
        return 0

    lax.fori_loop(0, nrows, body, 0)


def _make_deg_kernel(nchunk):
    @functools.partial(
        pl.kernel,
        out_type=jax.ShapeDtypeStruct((NC, N_PAD, DEGW), jnp.float32),
        mesh=_mesh,
        scratch_types=[
            pltpu.VMEM((nchunk, K), jnp.int32),
            pltpu.VMEM((K, DEGW), jnp.float32),
            pltpu.VMEM((RPT, DEGW), jnp.float32),
            pltpu.VMEM_SHARED((N_PAD, DEGW), jnp.float32),
            pltpu.SemaphoreType.DMA,
        ],
        compiler_params=pltpu.CompilerParams(use_tc_tiling_on_sc=False),
    )
    def deg_kernel(col_hbm, degp_hbm, col_v, ones_v, zrow_v, deg_sh, dsem):
        cid = lax.axis_index("c")
        sid = lax.axis_index("s")
        wid = sid * NC + cid

        pltpu.sync_copy(col_hbm.at[wid], col_v)

        one = jnp.full((16,), 1.0, jnp.float32)

        def fill_ones(i, _):
            ones_v[i, :] = one
            return 0

        lax.fori_loop(0, K, fill_ones, 0)
        _zero_rows(zrow_v, RPT, DEGW)

        pltpu.sync_copy(zrow_v, deg_sh.at[pl.ds(sid * RPT, RPT)])
        plsc.subcore_barrier()

        for j in range(8):
            pltpu.async_copy(ones_v, deg_sh.at[col_v.at[j]], dsem, add=True)

        def chunk(h, _):
            for j in range(8):
                s = 8 * h + j
                pltpu.make_async_copy(
                    ones_v, deg_sh.at[col_v.at[s]], dsem).wait()
                pltpu.async_copy(
                    ones_v, deg_sh.at[col_v.at[s + 8]], dsem, add=True)
            return 0

        lax.fori_loop(0, nchunk // 8 - 1, chunk, 0)
        for j in range(8):
            pltpu.make_async_copy(
                ones_v, deg_sh.at[col_v.at[j]], dsem).wait()
        plsc.subcore_barrier()

        pltpu.sync_copy(
            deg_sh.at[pl.ds(sid * RPT, RPT)],
            degp_hbm.at[cid, pl.ds(sid * RPT, RPT)],
        )

    return deg_kernel


def _make_agg_kernel(nchunk):
    @functools.partial(
        pl.kernel,
        out_type=jax.ShapeDtypeStruct((NC, N_PAD, HID), jnp.float32),
        mesh=_mesh,
        scratch_types=[
            pltpu.VMEM((nchunk, K), jnp.int32),
            pltpu.VMEM((nchunk, K), jnp.int32),
            [pltpu.VMEM((K, HID), jnp.float32) for _ in range(NBUF)],
            pltpu.VMEM((RPT, HID), jnp.float32),
            pltpu.VMEM_SHARED((N_PAD, HID), jnp.float32),
            pltpu.VMEM_SHARED((N_PAD, HID), jnp.float32),
            [pltpu.SemaphoreType.DMA for _ in range(NBUF)],
            [pltpu.SemaphoreType.DMA for _ in range(NBUF)],
        ],
        compiler_params=pltpu.CompilerParams(use_tc_tiling_on_sc=False),
    )
    def agg_kernel(feat_hbm, row_hbm, col_hbm, aggp_hbm,
                   row_v, col_v, bufs, zrow_v, agg_sh, feat_sh, gsem, ssem):
        cid = lax.axis_index("c")
        sid = lax.axis_index("s")
        wid = sid * NC + cid

        pltpu.sync_copy(row_hbm.at[wid], row_v)
        pltpu.sync_copy(col_hbm.at[wid], col_v)

        pltpu.sync_copy(
            feat_hbm.at[pl.ds(sid * RPT, RPT)],
            feat_sh.at[pl.ds(sid * RPT, RPT)],
        )

        _zero_rows(zrow_v, RPT, HID)
        pltpu.sync_copy(zrow_v, agg_sh.at[pl.ds(sid * RPT, RPT)])
        plsc.subcore_barrier()

        zsrc = zrow_v.at[pl.ds(0, K)]
        for j in range(HALF):
            pltpu.async_copy(
                zsrc, agg_sh.at[col_v.at[0]], ssem[HALF + j], add=True)
        for j in range(HALF):
            pltpu.async_copy(feat_sh.at[row_v.at[j]], bufs[j], gsem[j])

        def step(h, _):
            for j in range(NBUF):
                s = NBUF * h + j
                pltpu.make_async_copy(
                    feat_sh.at[row_v.at[s]], bufs[j], gsem[j]).wait()
                pltpu.async_copy(
                    bufs[j], agg_sh.at[col_v.at[s]], ssem[j], add=True)
                bb = (j + HALF) % NBUF
                pltpu.make_async_copy(
                    bufs[bb], agg_sh.at[col_v.at[s]], ssem[bb]).wait()
                s2 = (s + HALF) % nchunk
                pltpu.async_copy(feat_sh.at[row_v.at[s2]], bufs[bb], gsem[bb])
            return 0

        lax.fori_loop(0, nchunk // NBUF, step, 0)

        for j in range(HALF):
            pltpu.make_async_copy(
                bufs[HALF + j], agg_sh.at[col_v.at[0]], ssem[HALF + j]).wait()
            pltpu.make_async_copy(
                feat_sh.at[row_v.at[j]], bufs[j], gsem[j]).wait()
        plsc.subcore_barrier()

        pltpu.sync_copy(
            agg_sh.at[pl.ds(sid * RPT, RPT)],
            aggp_hbm.at[cid, pl.ds(sid * RPT, RPT)],
        )

    return agg_kernel


def _dinv_from_partials(degp):
    deg = degp[0, :, 0:1] + degp[1, :, 0:1]
    return jnp.where(deg > 0, lax.rsqrt(jnp.maximum(deg, 1e-12)), 0.0)


def _tc_xw_scale(x_pad, w1, degp):
    def body(x_ref, w_ref, degp_ref, out_ref):
        dinv = _dinv_from_partials(degp_ref[...])
        xw = jnp.dot(x_ref[...], w_ref[...],
                     preferred_element_type=jnp.float32)
        out_ref[...] = xw * dinv

    return pl.pallas_call(
        body,
        out_shape=jax.ShapeDtypeStruct((N_PAD, HID), jnp.float32),
    )(x_pad, w1, degp)


def _tc_mid(aggp, degp, b1):
    def body(aggp_ref, degp_ref, b1_ref, out_ref):
        dinv = _dinv_from_partials(degp_ref[...])
        agg = aggp_ref[0] + aggp_ref[1]
        h = jnp.maximum(agg * dinv + b1_ref[...], 0.0)
        rows = lax.broadcasted_iota(jnp.int32, (N_PAD, 1), 0)
        out_ref[...] = jnp.where(rows < N, h * dinv, 0.0)

    return pl.pallas_call(
        body,
        out_shape=jax.ShapeDtypeStruct((N_PAD, HID), jnp.float32),
    )(aggp, degp, b1)


def _tc_final(aggp, degp, w2, b2):
    def body(aggp_ref, degp_ref, w2_ref, b2_ref, out_ref):
        dinv = _dinv_from_partials(degp_ref[...])
        agg = (aggp_ref[0] + aggp_ref[1]) * dinv
        y = jnp.dot(agg[:N, :], w2_ref[...],
                    preferred_element_type=jnp.float32) + b2_ref[...]
        m = jnp.max(y, axis=1, keepdims=True)
        s = y - m
        lse = jnp.log(jnp.sum(jnp.exp(s), axis=1, keepdims=True))
        out_ref[...] = s - lse

    return pl.pallas_call(
        body,
        out_shape=jax.ShapeDtypeStruct((N, 128), jnp.float32),
    )(aggp, degp, w2, b2)


def kernel(x, edge_index, W1, b1, W2, b2):
    e = edge_index.shape[1]
    nchunk = -(-e // (NW * K))
    nchunk = -(-nchunk // NBUF) * NBUF
    e_pad = NW * nchunk * K

    row = edge_index[0].astype(jnp.int32)
    col = edge_index[1].astype(jnp.int32)
    pad = jnp.full((e_pad - e,), N, jnp.int32)
    row_b = jnp.concatenate([row, pad]).reshape(NW, nchunk, K)
    col_b = jnp.concatenate([col, pad]).reshape(NW, nchunk, K)

    x_pad = jnp.pad(x, ((0, N_PAD - x.shape[0]), (0, 0)))

    degp = _make_deg_kernel(nchunk)(col_b)
    t1s = _tc_xw_scale(x_pad, W1, degp)
    agg1p = _make_agg_kernel(nchunk)(t1s, row_b, col_b)
    h_s = _tc_mid(agg1p, degp, b1)
    agg2p = _make_agg_kernel(nchunk)(h_s, row_b, col_b)
    return _tc_final(agg2p, degp, W2, b2)
